# baseline (speedup 1.0000x reference)
import functools

import jax
import jax.numpy as jnp
from jax import lax
from jax.experimental import pallas as pl
from jax.experimental.pallas import tpu as pltpu
from jax.experimental.pallas import tpu_sc as plsc

N_NODES = 10000
N_EDGES = 320000
FEAT = 128
N_GRAPHS = 64

NC, NS = 2, 16
NW = NC * NS
EPW = N_EDGES // NW
CH = 80
NCHUNK = EPW // CH
N_PAD = 10240
RPT = N_PAD // NS
ZR = 128
VEC = 16


def _sc_aggr_body(h_hbm, e_hbm, src_hbm, dst_hbm, out_hbm,
                  aggr_sh, src_v, dst_v, xrows_v, erows_v, zero_v, sem):
    cid = lax.axis_index("c")
    sid = lax.axis_index("s")
    wid = cid * NS + sid

    z = jnp.zeros((VEC,), jnp.float32)

    def zrow(r, carry):
        for c in range(FEAT // VEC):
            zero_v[r, pl.ds(c * VEC, VEC)] = z
        return carry
    lax.fori_loop(0, ZR, zrow, 0)

    for k in range(RPT // ZR):
        pltpu.sync_copy(zero_v, aggr_sh.at[pl.ds(sid * RPT + k * ZR, ZR)])
    plsc.subcore_barrier()

    def chunk(t, carry):
        eb = wid * EPW + t * CH
        pltpu.sync_copy(src_hbm.at[pl.ds(eb, CH)], src_v)
        pltpu.sync_copy(dst_hbm.at[pl.ds(eb, CH)], dst_v)
        cp = pltpu.async_copy(h_hbm.at[src_v], xrows_v, sem)
        pltpu.sync_copy(e_hbm.at[pl.ds(eb, CH)], erows_v)
        cp.wait()

        def row(r, c2):
            for c in range(FEAT // VEC):
                sl = pl.ds(c * VEC, VEC)
                erows_v[r, sl] = jnp.maximum(xrows_v[r, sl] + erows_v[r, sl],
                                             0.0)
            return c2
        lax.fori_loop(0, CH, row, 0, unroll=4)

        pltpu.sync_copy(erows_v, aggr_sh.at[dst_v], add=True)
        return carry
    lax.fori_loop(0, NCHUNK, chunk, 0)

    plsc.subcore_barrier()
    pltpu.sync_copy(aggr_sh.at[pl.ds(sid * RPT, RPT)],
                    out_hbm.at[cid, pl.ds(sid * RPT, RPT)])


@functools.cache
def _make_sc_aggr():
    return pl.kernel(
        _sc_aggr_body,
        out_type=jax.ShapeDtypeStruct((NC, N_PAD, FEAT), jnp.float32),
        mesh=plsc.VectorSubcoreMesh(core_axis_name="c", subcore_axis_name="s",
                                    num_cores=NC, num_subcores=NS),
        scratch_types=[
            pltpu.VMEM_SHARED((N_PAD, FEAT), jnp.float32),
            pltpu.VMEM((CH,), jnp.int32),
            pltpu.VMEM((CH,), jnp.int32),
            pltpu.VMEM((CH, FEAT), jnp.float32),
            pltpu.VMEM((CH, FEAT), jnp.float32),
            pltpu.VMEM((ZR, FEAT), jnp.float32),
            pltpu.SemaphoreType.DMA,
        ],
    )


def _sc_aggr(h, e, src, dst):
    return _make_sc_aggr()(h, e, src, dst)[:, :N_NODES, :]


def _edge_mlp_body(ea_ref, w_ref, b_ref, out_ref):
    out_ref[...] = jnp.dot(ea_ref[...], w_ref[...],
                           preferred_element_type=jnp.float32) + b_ref[...]


_EB = 2000


def _edge_mlp(ea, W, b):
    ed = ea.shape[1]
    return pl.pallas_call(
        _edge_mlp_body,
        grid=(N_EDGES // _EB,),
        in_specs=[
            pl.BlockSpec((_EB, ed), lambda i: (i, 0)),
            pl.BlockSpec((ed, FEAT), lambda i: (0, 0)),
            pl.BlockSpec((1, FEAT), lambda i: (0, 0)),
        ],
        out_specs=pl.BlockSpec((_EB, FEAT), lambda i: (i, 0)),
        out_shape=jax.ShapeDtypeStruct((N_EDGES, FEAT), jnp.float32),
    )(ea, W, b.reshape(1, FEAT))


def _node_body(h_ref, a_ref, w_ref, b_ref, out_ref):
    s = h_ref[...] + a_ref[0] + a_ref[1]
    out_ref[...] = jax.nn.gelu(
        jnp.dot(s, w_ref[...], preferred_element_type=jnp.float32)
        + b_ref[...])


_NB = 2000


def _node_update(h, parts, W, b):
    return pl.pallas_call(
        _node_body,
        grid=(N_NODES // _NB,),
        in_specs=[
            pl.BlockSpec((_NB, FEAT), lambda i: (i, 0)),
            pl.BlockSpec((NC, _NB, FEAT), lambda i: (0, i, 0)),
            pl.BlockSpec((FEAT, FEAT), lambda i: (0, 0)),
            pl.BlockSpec((1, FEAT), lambda i: (0, 0)),
        ],
        out_specs=pl.BlockSpec((_NB, FEAT), lambda i: (i, 0)),
        out_shape=jax.ShapeDtypeStruct((N_NODES, FEAT), jnp.float32),
    )(h, parts, W, b.reshape(1, FEAT))


def _head_body(h_ref, batch_ref, w1_ref, b1_ref, w2_ref, b2_ref, out_ref):
    onehot = (batch_ref[...] ==
              lax.broadcasted_iota(jnp.int32, (1, N_GRAPHS), 1)
              ).astype(jnp.float32)
    sums = lax.dot_general(onehot, h_ref[...], (((0,), (0,)), ((), ())),
                           preferred_element_type=jnp.float32)
    counts = jnp.sum(onehot, axis=0)
    pooled = sums / jnp.maximum(counts, 1.0)[:, None]
    t = jax.nn.gelu(jnp.dot(pooled, w1_ref[...],
                            preferred_element_type=jnp.float32) + b1_ref[...])
    out_ref[...] = jnp.dot(t, w2_ref[...],
                           preferred_element_type=jnp.float32) + b2_ref[...]


def _head(h, batch, fc1_W, fc1_b, fc2_W, fc2_b):
    return pl.pallas_call(
        _head_body,
        out_shape=jax.ShapeDtypeStruct((N_GRAPHS, 1), jnp.float32),
    )(h, batch.reshape(N_NODES, 1), fc1_W, fc1_b.reshape(1, 64),
      fc2_W, fc2_b.reshape(1, 1))


def kernel(x, edge_index, batch, edge_attr,
           We0, be0, Wn0, bn0, We1, be1, Wn1, bn1, We2, be2, Wn2, bn2,
           fc1_W, fc1_b, fc2_W, fc2_b):
    src = edge_index[0]
    dst = edge_index[1]
    h = x
    for We, be, Wn, bn in ((We0, be0, Wn0, bn0),
                           (We1, be1, Wn1, bn1),
                           (We2, be2, Wn2, bn2)):
        e = _edge_mlp(edge_attr, We, be)
        parts = _sc_aggr(h, e, src, dst)
        h = _node_update(h, parts, Wn, bn)
    return _head(h, batch, fc1_W, fc1_b, fc2_W, fc2_b)


# speedup vs baseline: 1.5451x; 1.5451x over previous
import functools

import jax
import jax.numpy as jnp
from jax import lax
from jax.experimental import pallas as pl
from jax.experimental.pallas import tpu as pltpu
from jax.experimental.pallas import tpu_sc as plsc

N_NODES = 10000
N_EDGES = 320000
FEAT = 128
N_GRAPHS = 64

NC, NS = 2, 16
NW = NC * NS
EPW = N_EDGES // NW
CH = 80
NCHUNK = EPW // CH
N_PAD = 10240
RPT = N_PAD // NS
ZR = 128
VEC = 16


def _sc_aggr_body(h_hbm, e_hbm, src_hbm, dst_hbm, out_hbm,
                  aggr_sh, src_v, dst_v, xrows_v, erows_v, zero_v, sem):
    cid = lax.axis_index("c")
    sid = lax.axis_index("s")
    wid = cid * NS + sid

    z = jnp.zeros((VEC,), jnp.float32)

    def zrow(r, carry):
        for c in range(FEAT // VEC):
            zero_v[r, pl.ds(c * VEC, VEC)] = z
        return carry
    lax.fori_loop(0, ZR, zrow, 0)

    for k in range(RPT // ZR):
        pltpu.sync_copy(zero_v, aggr_sh.at[pl.ds(sid * RPT + k * ZR, ZR)])
    plsc.subcore_barrier()

    def chunk(t, carry):
        eb = wid * EPW + t * CH
        pltpu.sync_copy(src_hbm.at[pl.ds(eb, CH)], src_v)
        pltpu.sync_copy(dst_hbm.at[pl.ds(eb, CH)], dst_v)
        cp = pltpu.async_copy(h_hbm.at[src_v], xrows_v, sem)
        pltpu.sync_copy(e_hbm.at[pl.ds(eb, CH)], erows_v)
        cp.wait()

        def row(r, c2):
            for c in range(FEAT // VEC):
                sl = pl.ds(c * VEC, VEC)
                erows_v[r, sl] = jnp.maximum(xrows_v[r, sl] + erows_v[r, sl],
                                             0.0)
            return c2
        lax.fori_loop(0, CH, row, 0)

        pltpu.sync_copy(erows_v, aggr_sh.at[dst_v], add=True)
        return carry
    lax.fori_loop(0, NCHUNK, chunk, 0)

    plsc.subcore_barrier()
    pltpu.sync_copy(aggr_sh.at[pl.ds(sid * RPT, RPT)],
                    out_hbm.at[cid, pl.ds(sid * RPT, RPT)])


@functools.cache
def _make_sc_aggr():
    return pl.kernel(
        _sc_aggr_body,
        out_type=jax.ShapeDtypeStruct((NC, N_PAD, FEAT), jnp.float32),
        mesh=plsc.VectorSubcoreMesh(core_axis_name="c", subcore_axis_name="s",
                                    num_cores=NC, num_subcores=NS),
        scratch_types=[
            pltpu.VMEM_SHARED((N_PAD, FEAT), jnp.float32),
            pltpu.VMEM((CH,), jnp.int32),
            pltpu.VMEM((CH,), jnp.int32),
            pltpu.VMEM((CH, FEAT), jnp.float32),
            pltpu.VMEM((CH, FEAT), jnp.float32),
            pltpu.VMEM((ZR, FEAT), jnp.float32),
            pltpu.SemaphoreType.DMA,
        ],
    )


def _sc_aggr(h, e, src, dst):
    return _make_sc_aggr()(h, e, src, dst)[:, :N_NODES, :]


def _edge_mlp_body(ea_ref, w_ref, b_ref, out_ref):
    out_ref[...] = jnp.dot(ea_ref[...], w_ref[...],
                           preferred_element_type=jnp.float32) + b_ref[...]


_EB = 2000


def _edge_mlp(ea, W, b):
    ed = ea.shape[1]
    return pl.pallas_call(
        _edge_mlp_body,
        grid=(N_EDGES // _EB,),
        in_specs=[
            pl.BlockSpec((_EB, ed), lambda i: (i, 0)),
            pl.BlockSpec((ed, FEAT), lambda i: (0, 0)),
            pl.BlockSpec((1, FEAT), lambda i: (0, 0)),
        ],
        out_specs=pl.BlockSpec((_EB, FEAT), lambda i: (i, 0)),
        out_shape=jax.ShapeDtypeStruct((N_EDGES, FEAT), jnp.float32),
    )(ea, W, b.reshape(1, FEAT))


def _node_body(h_ref, a_ref, w_ref, b_ref, out_ref):
    s = h_ref[...] + a_ref[0] + a_ref[1]
    out_ref[...] = jax.nn.gelu(
        jnp.dot(s, w_ref[...], preferred_element_type=jnp.float32)
        + b_ref[...])


_NB = 2000


def _node_update(h, parts, W, b):
    return pl.pallas_call(
        _node_body,
        grid=(N_NODES // _NB,),
        in_specs=[
            pl.BlockSpec((_NB, FEAT), lambda i: (i, 0)),
            pl.BlockSpec((NC, _NB, FEAT), lambda i: (0, i, 0)),
            pl.BlockSpec((FEAT, FEAT), lambda i: (0, 0)),
            pl.BlockSpec((1, FEAT), lambda i: (0, 0)),
        ],
        out_specs=pl.BlockSpec((_NB, FEAT), lambda i: (i, 0)),
        out_shape=jax.ShapeDtypeStruct((N_NODES, FEAT), jnp.float32),
    )(h, parts, W, b.reshape(1, FEAT))


def _head_body(h_ref, batch_ref, w1_ref, b1_ref, w2_ref, b2_ref, out_ref):
    onehot = (batch_ref[...] ==
              lax.broadcasted_iota(jnp.int32, (1, N_GRAPHS), 1)
              ).astype(jnp.float32)
    sums = lax.dot_general(onehot, h_ref[...], (((0,), (0,)), ((), ())),
                           preferred_element_type=jnp.float32)
    counts = jnp.sum(onehot, axis=0)
    pooled = sums / jnp.maximum(counts, 1.0)[:, None]
    t = jax.nn.gelu(jnp.dot(pooled, w1_ref[...],
                            preferred_element_type=jnp.float32) + b1_ref[...])
    out_ref[...] = jnp.dot(t, w2_ref[...],
                           preferred_element_type=jnp.float32) + b2_ref[...]


def _head(h, batch, fc1_W, fc1_b, fc2_W, fc2_b):
    return pl.pallas_call(
        _head_body,
        out_shape=jax.ShapeDtypeStruct((N_GRAPHS, 1), jnp.float32),
    )(h, batch.reshape(N_NODES, 1), fc1_W, fc1_b.reshape(1, 64),
      fc2_W, fc2_b.reshape(1, 1))


def kernel(x, edge_index, batch, edge_attr,
           We0, be0, Wn0, bn0, We1, be1, Wn1, bn1, We2, be2, Wn2, bn2,
           fc1_W, fc1_b, fc2_W, fc2_b):
    src = edge_index[0]
    dst = edge_index[1]
    h = x
    for We, be, Wn, bn in ((We0, be0, Wn0, bn0),
                           (We1, be1, Wn1, bn1),
                           (We2, be2, Wn2, bn2)):
        e = _edge_mlp(edge_attr, We, be)
        parts = _sc_aggr(h, e, src, dst)
        h = _node_update(h, parts, Wn, bn)
    return _head(h, batch, fc1_W, fc1_b, fc2_W, fc2_b)
